# baseline (speedup 1.0000x reference)
import functools

import jax
import jax.numpy as jnp
from jax import lax
from jax.experimental import pallas as pl
from jax.experimental.pallas import tpu as pltpu
from jax.experimental.pallas import tpu_sc as plsc

NUM_BINS = 256
LANES = 16
CHUNK = 32768
UNROLL = 32
NPAIR = 6
PAD = 16
BANK = PAD + LANES * NUM_BINS


def _make_kernel(rows, cols):
    n_chunks = cols // CHUNK
    assert n_chunks % 2 == 0
    mesh = plsc.VectorSubcoreMesh(core_axis_name="c", subcore_axis_name="s")

    @functools.partial(
        pl.kernel,
        out_type=jax.ShapeDtypeStruct((rows, NUM_BINS), jnp.float32),
        mesh=mesh,
        scratch_types=[
            pltpu.VMEM((CHUNK,), jnp.float32),
            pltpu.VMEM((CHUNK,), jnp.float32),
        ] + [pltpu.VMEM((BANK,), jnp.float32)] * (2 * NPAIR) + [
            pltpu.VMEM((NUM_BINS,), jnp.float32),
            pltpu.SemaphoreType.DMA,
            pltpu.SemaphoreType.DMA,
        ],
        compiler_params=pltpu.CompilerParams(needs_layout_passes=False),
    )
    def hist_kernel(x_hbm, out_hbm, buf0, buf1, *rest):
        c_banks = rest[:NPAIR]
        s_banks = rest[NPAIR:2 * NPAIR]
        row_buf, sem0, sem1 = rest[2 * NPAIR:]
        row = lax.axis_index("s") * mesh.num_cores + lax.axis_index("c")

        zeros = jnp.zeros((LANES,), jnp.float32)
        ones = jnp.ones((LANES,), jnp.float32)

        def zero_body(i, _):
            for a in c_banks + s_banks:
                a[pl.ds(i * LANES, LANES)] = zeros
            return 0
        lax.fori_loop(0, BANK // LANES, zero_body, 0)

        lane_pad = lax.iota(jnp.int32, LANES) + PAD

        def start_dma(k, buf, sem):
            return pltpu.async_copy(
                x_hbm.at[row, pl.ds(k * CHUNK, CHUNK)], buf, sem)

        def wait_dma(buf, sem):
            pltpu.make_async_copy(
                x_hbm.at[row, pl.ds(0, CHUNK)], buf, sem).wait()

        def do_scatters(idxs, fracs):
            for j in range(UNROLL):
                plsc.addupdate_scatter(c_banks[j % NPAIR], [idxs[j]], ones)
                plsc.addupdate_scatter(s_banks[j % NPAIR], [idxs[j]],
                                       fracs[j])

        def process_chunk(buf):
            def body(i, _):
                base = i * (LANES * UNROLL)
                vs = [buf[pl.ds(base + j * LANES, LANES)]
                      for j in range(UNROLL)]
                ibs = [v.astype(jnp.int32) for v in vs]
                idxs = [(ib << 4) + lane_pad for ib in ibs]
                fracs = [v - ib.astype(jnp.float32)
                         for v, ib in zip(vs, ibs)]
                do_scatters(idxs, fracs)
                return 0
            lax.fori_loop(0, CHUNK // (LANES * UNROLL), body, 0)

        start_dma(0, buf0, sem0)
        start_dma(1, buf1, sem1)

        def chunk_body(g, _):
            k = 2 * g
            wait_dma(buf0, sem0)
            process_chunk(buf0)

            @pl.when(k + 2 < n_chunks)
            def _():
                start_dma(k + 2, buf0, sem0)

            wait_dma(buf1, sem1)
            process_chunk(buf1)

            @pl.when(k + 3 < n_chunks)
            def _():
                start_dma(k + 3, buf1, sem1)
            return 0
        lax.fori_loop(0, n_chunks // 2, chunk_body, 0)

        lane0 = lax.iota(jnp.int32, LANES) == 0

        def reduce_body(j, carry):
            s_prev = carry
            c_cur = zeros
            s_cur = zeros
            off = PAD + j * LANES
            for cb in c_banks:
                c_cur = c_cur + cb[pl.ds(off, LANES)]
            for sb in s_banks:
                s_cur = s_cur + sb[pl.ds(off, LANES)]
            s = c_cur - s_cur + s_prev
            total = jnp.full((LANES,), jnp.sum(s))
            jidx = jnp.full((LANES,), j, jnp.int32)
            plsc.store_scatter(row_buf, [jidx], total, mask=lane0)
            return s_cur
        lax.fori_loop(0, NUM_BINS, reduce_body, zeros)

        pltpu.sync_copy(row_buf, out_hbm.at[row])

    return hist_kernel


@jax.jit
def kernel(x):
    rows, cols = x.shape
    return _make_kernel(rows, cols)(x)


# speedup vs baseline: 1.0118x; 1.0118x over previous
import functools

import jax
import jax.numpy as jnp
from jax import lax
from jax.experimental import pallas as pl
from jax.experimental.pallas import tpu as pltpu
from jax.experimental.pallas import tpu_sc as plsc

NUM_BINS = 256
LANES = 16
CHUNK = 32768
UNROLL = 16
NPAIR = 6
PAD = 16
BANK = PAD + LANES * NUM_BINS


def _make_kernel(rows, cols):
    n_chunks = cols // CHUNK
    assert n_chunks % 2 == 0
    mesh = plsc.VectorSubcoreMesh(core_axis_name="c", subcore_axis_name="s")

    @functools.partial(
        pl.kernel,
        out_type=jax.ShapeDtypeStruct((rows, NUM_BINS), jnp.float32),
        mesh=mesh,
        scratch_types=[
            pltpu.VMEM((CHUNK,), jnp.float32),
            pltpu.VMEM((CHUNK,), jnp.float32),
        ] + [pltpu.VMEM((BANK,), jnp.float32)] * (2 * NPAIR) + [
            pltpu.VMEM((NUM_BINS,), jnp.float32),
            pltpu.SemaphoreType.DMA,
            pltpu.SemaphoreType.DMA,
        ],
        compiler_params=pltpu.CompilerParams(needs_layout_passes=False),
    )
    def hist_kernel(x_hbm, out_hbm, buf0, buf1, *rest):
        c_banks = rest[:NPAIR]
        s_banks = rest[NPAIR:2 * NPAIR]
        row_buf, sem0, sem1 = rest[2 * NPAIR:]
        row = lax.axis_index("s") * mesh.num_cores + lax.axis_index("c")

        zeros = jnp.zeros((LANES,), jnp.float32)
        ones = jnp.ones((LANES,), jnp.float32)

        def zero_body(i, _):
            for a in c_banks + s_banks:
                a[pl.ds(i * 2 * LANES, LANES)] = zeros
                a[pl.ds(i * 2 * LANES + LANES, LANES)] = zeros
            return 0
        lax.fori_loop(0, BANK // (2 * LANES), zero_body, 0)

        lane_pad = lax.iota(jnp.int32, LANES) + PAD

        def start_dma(k, buf, sem):
            return pltpu.async_copy(
                x_hbm.at[row, pl.ds(k * CHUNK, CHUNK)], buf, sem)

        def wait_dma(buf, sem):
            pltpu.make_async_copy(
                x_hbm.at[row, pl.ds(0, CHUNK)], buf, sem).wait()

        def do_scatters(idxs, fracs):
            for j in range(UNROLL):
                plsc.addupdate_scatter(c_banks[j % NPAIR], [idxs[j]], ones)
                plsc.addupdate_scatter(s_banks[j % NPAIR], [idxs[j]],
                                       fracs[j])

        def process_chunk(buf):
            def body(i, _):
                base = i * (LANES * UNROLL)
                vs = [buf[pl.ds(base + j * LANES, LANES)]
                      for j in range(UNROLL)]
                ibs = [v.astype(jnp.int32) for v in vs]
                idxs = [(ib << 4) + lane_pad for ib in ibs]
                fracs = [v - ib.astype(jnp.float32)
                         for v, ib in zip(vs, ibs)]
                do_scatters(idxs, fracs)
                return 0
            lax.fori_loop(0, CHUNK // (LANES * UNROLL), body, 0)

        start_dma(0, buf0, sem0)
        start_dma(1, buf1, sem1)

        def chunk_body(g, _):
            k = 2 * g
            wait_dma(buf0, sem0)
            process_chunk(buf0)

            @pl.when(k + 2 < n_chunks)
            def _():
                start_dma(k + 2, buf0, sem0)

            wait_dma(buf1, sem1)
            process_chunk(buf1)

            @pl.when(k + 3 < n_chunks)
            def _():
                start_dma(k + 3, buf1, sem1)
            return 0
        lax.fori_loop(0, n_chunks // 2, chunk_body, 0)

        lane0 = lax.iota(jnp.int32, LANES) == 0

        def reduce_body(i, carry):
            s_prev = carry
            j0 = i * 2
            sums = []
            for d in range(2):
                c_cur = zeros
                s_cur = zeros
                off = PAD + (j0 + d) * LANES
                for cb in c_banks:
                    c_cur = c_cur + cb[pl.ds(off, LANES)]
                for sb in s_banks:
                    s_cur = s_cur + sb[pl.ds(off, LANES)]
                sums.append(jnp.sum(c_cur - s_cur + s_prev))
                s_prev = s_cur
            for d in range(2):
                total = jnp.full((LANES,), sums[d])
                jidx = jnp.full((LANES,), j0 + d, jnp.int32)
                plsc.store_scatter(row_buf, [jidx], total, mask=lane0)
            return s_prev
        lax.fori_loop(0, NUM_BINS // 2, reduce_body, zeros)

        pltpu.sync_copy(row_buf, out_hbm.at[row])

    return hist_kernel


@jax.jit
def kernel(x):
    rows, cols = x.shape
    return _make_kernel(rows, cols)(x)


# speedup vs baseline: 1.0122x; 1.0004x over previous
import functools

import jax
import jax.numpy as jnp
from jax import lax
from jax.experimental import pallas as pl
from jax.experimental.pallas import tpu as pltpu
from jax.experimental.pallas import tpu_sc as plsc

NUM_BINS = 256
LANES = 16
CHUNK = 32768
UNROLL = 16
NPAIR = 6
PAD = 16
BANK = PAD + LANES * NUM_BINS + 16


def _make_kernel(rows, cols):
    n_chunks = cols // CHUNK
    assert n_chunks % 2 == 0
    mesh = plsc.VectorSubcoreMesh(core_axis_name="c", subcore_axis_name="s")

    @functools.partial(
        pl.kernel,
        out_type=jax.ShapeDtypeStruct((rows, NUM_BINS), jnp.float32),
        mesh=mesh,
        scratch_types=[
            pltpu.VMEM((CHUNK,), jnp.float32),
            pltpu.VMEM((CHUNK,), jnp.float32),
        ] + [pltpu.VMEM((BANK,), jnp.float32)] * (2 * NPAIR) + [
            pltpu.VMEM((NUM_BINS,), jnp.float32),
            pltpu.SemaphoreType.DMA,
            pltpu.SemaphoreType.DMA,
        ],
        compiler_params=pltpu.CompilerParams(needs_layout_passes=False),
    )
    def hist_kernel(x_hbm, out_hbm, buf0, buf1, *rest):
        c_banks = rest[:NPAIR]
        s_banks = rest[NPAIR:2 * NPAIR]
        row_buf, sem0, sem1 = rest[2 * NPAIR:]
        row = lax.axis_index("s") * mesh.num_cores + lax.axis_index("c")

        zeros = jnp.zeros((LANES,), jnp.float32)
        ones = jnp.ones((LANES,), jnp.float32)

        def zero_body(i, _):
            for a in c_banks + s_banks:
                a[pl.ds(i * 2 * LANES, LANES)] = zeros
                a[pl.ds(i * 2 * LANES + LANES, LANES)] = zeros
            return 0
        lax.fori_loop(0, BANK // (2 * LANES), zero_body, 0)

        lane_pad = lax.iota(jnp.int32, LANES) + PAD

        def start_dma(k, buf, sem):
            return pltpu.async_copy(
                x_hbm.at[row, pl.ds(k * CHUNK, CHUNK)], buf, sem)

        def wait_dma(buf, sem):
            pltpu.make_async_copy(
                x_hbm.at[row, pl.ds(0, CHUNK)], buf, sem).wait()

        def do_scatters(idxs, fracs):
            for j in range(UNROLL):
                plsc.addupdate_scatter(c_banks[j % NPAIR], [idxs[j]], ones)
                plsc.addupdate_scatter(s_banks[j % NPAIR], [idxs[j]],
                                       fracs[j])

        def process_chunk(buf):
            def body(i, _):
                base = i * (LANES * UNROLL)
                vs = [buf[pl.ds(base + j * LANES, LANES)]
                      for j in range(UNROLL)]
                ibs = [v.astype(jnp.int32) for v in vs]
                idxs = [(ib << 4) + lane_pad for ib in ibs]
                fracs = [v - ib.astype(jnp.float32)
                         for v, ib in zip(vs, ibs)]
                do_scatters(idxs, fracs)
                return 0
            lax.fori_loop(0, CHUNK // (LANES * UNROLL), body, 0)

        start_dma(0, buf0, sem0)
        start_dma(1, buf1, sem1)

        def chunk_body(g, _):
            k = 2 * g
            wait_dma(buf0, sem0)
            process_chunk(buf0)

            @pl.when(k + 2 < n_chunks)
            def _():
                start_dma(k + 2, buf0, sem0)

            wait_dma(buf1, sem1)
            process_chunk(buf1)

            @pl.when(k + 3 < n_chunks)
            def _():
                start_dma(k + 3, buf1, sem1)
            return 0
        lax.fori_loop(0, n_chunks // 2, chunk_body, 0)

        lane0 = lax.iota(jnp.int32, LANES) == 0

        def reduce_body(i, carry):
            s_prev = carry
            j0 = i * 2
            sums = []
            for d in range(2):
                c_cur = zeros
                s_cur = zeros
                off = PAD + (j0 + d) * LANES
                for cb in c_banks:
                    c_cur = c_cur + cb[pl.ds(off, LANES)]
                for sb in s_banks:
                    s_cur = s_cur + sb[pl.ds(off, LANES)]
                sums.append(jnp.sum(c_cur - s_cur + s_prev))
                s_prev = s_cur
            for d in range(2):
                total = jnp.full((LANES,), sums[d])
                jidx = jnp.full((LANES,), j0 + d, jnp.int32)
                plsc.store_scatter(row_buf, [jidx], total, mask=lane0)
            return s_prev
        lax.fori_loop(0, NUM_BINS // 2, reduce_body, zeros)

        pltpu.sync_copy(row_buf, out_hbm.at[row])

    return hist_kernel


@jax.jit
def kernel(x):
    rows, cols = x.shape
    return _make_kernel(rows, cols)(x)


# speedup vs baseline: 1.0232x; 1.0109x over previous
import functools

import jax
import jax.numpy as jnp
from jax import lax
from jax.experimental import pallas as pl
from jax.experimental.pallas import tpu as pltpu
from jax.experimental.pallas import tpu_sc as plsc

NUM_BINS = 256
LANES = 16
CHUNK = 32768
UNROLL = 16
NPAIR = 6
PAD = 16
BANK = PAD + LANES * NUM_BINS + 16


def _make_kernel(rows, cols):
    n_chunks = cols // CHUNK
    assert n_chunks % 2 == 0
    mesh = plsc.VectorSubcoreMesh(core_axis_name="c", subcore_axis_name="s")

    @functools.partial(
        pl.kernel,
        out_type=jax.ShapeDtypeStruct((rows, NUM_BINS), jnp.float32),
        mesh=mesh,
        scratch_types=[
            pltpu.VMEM((CHUNK,), jnp.float32),
            pltpu.VMEM((CHUNK,), jnp.float32),
        ] + [pltpu.VMEM((BANK,), jnp.float32)] * (2 * NPAIR) + [
            pltpu.VMEM((NUM_BINS,), jnp.float32),
            pltpu.SemaphoreType.DMA,
            pltpu.SemaphoreType.DMA,
        ],
        compiler_params=pltpu.CompilerParams(needs_layout_passes=False),
    )
    def hist_kernel(x_hbm, out_hbm, buf0, buf1, *rest):
        c_banks = rest[:NPAIR]
        s_banks = rest[NPAIR:2 * NPAIR]
        row_buf, sem0, sem1 = rest[2 * NPAIR:]
        row = lax.axis_index("s") * mesh.num_cores + lax.axis_index("c")

        zeros = jnp.zeros((LANES,), jnp.float32)
        ones = jnp.ones((LANES,), jnp.float32)

        def start_dma(k, buf, sem):
            return pltpu.async_copy(
                x_hbm.at[row, pl.ds(k * CHUNK, CHUNK)], buf, sem)

        start_dma(0, buf0, sem0)
        start_dma(1, buf1, sem1)

        def zero_body(i, _):
            for a in c_banks + s_banks:
                a[pl.ds(i * 2 * LANES, LANES)] = zeros
                a[pl.ds(i * 2 * LANES + LANES, LANES)] = zeros
            return 0
        lax.fori_loop(0, BANK // (2 * LANES), zero_body, 0)

        lane_pad = lax.iota(jnp.int32, LANES) + PAD

        def wait_dma(buf, sem):
            pltpu.make_async_copy(
                x_hbm.at[row, pl.ds(0, CHUNK)], buf, sem).wait()

        def do_scatters(idxs, fracs):
            for j in range(UNROLL):
                plsc.addupdate_scatter(c_banks[j % NPAIR], [idxs[j]], ones)
                plsc.addupdate_scatter(s_banks[j % NPAIR], [idxs[j]],
                                       fracs[j])

        def process_chunk(buf):
            def body(i, _):
                base = i * (LANES * UNROLL)
                vs = [buf[pl.ds(base + j * LANES, LANES)]
                      for j in range(UNROLL)]
                ibs = [v.astype(jnp.int32) for v in vs]
                idxs = [(ib << 4) + lane_pad for ib in ibs]
                fracs = [v - ib.astype(jnp.float32)
                         for v, ib in zip(vs, ibs)]
                do_scatters(idxs, fracs)
                return 0
            lax.fori_loop(0, CHUNK // (LANES * UNROLL), body, 0)

        def chunk_body(g, _):
            k = 2 * g
            wait_dma(buf0, sem0)
            process_chunk(buf0)

            @pl.when(k + 2 < n_chunks)
            def _():
                start_dma(k + 2, buf0, sem0)

            wait_dma(buf1, sem1)
            process_chunk(buf1)

            @pl.when(k + 3 < n_chunks)
            def _():
                start_dma(k + 3, buf1, sem1)
            return 0
        lax.fori_loop(0, n_chunks // 2, chunk_body, 0)

        lane0 = lax.iota(jnp.int32, LANES) == 0

        def reduce_body(i, carry):
            s_prev = carry
            j0 = i * 2
            sums = []
            for d in range(2):
                c_cur = zeros
                s_cur = zeros
                off = PAD + (j0 + d) * LANES
                for cb in c_banks:
                    c_cur = c_cur + cb[pl.ds(off, LANES)]
                for sb in s_banks:
                    s_cur = s_cur + sb[pl.ds(off, LANES)]
                sums.append(jnp.sum(c_cur - s_cur + s_prev))
                s_prev = s_cur
            for d in range(2):
                total = jnp.full((LANES,), sums[d])
                jidx = jnp.full((LANES,), j0 + d, jnp.int32)
                plsc.store_scatter(row_buf, [jidx], total, mask=lane0)
            return s_prev
        lax.fori_loop(0, NUM_BINS // 2, reduce_body, zeros)

        pltpu.sync_copy(row_buf, out_hbm.at[row])

    return hist_kernel


@jax.jit
def kernel(x):
    rows, cols = x.shape
    return _make_kernel(rows, cols)(x)
